# baseline (speedup 1.0000x reference)
import functools

import jax
import jax.numpy as jnp
from jax import lax
from jax.experimental import pallas as pl
from jax.experimental.pallas import tpu as pltpu
from jax.experimental.pallas import tpu_sc as plsc

BATCH = 16384
FIELDS = 26
DIM = 64
VOCAB = 1000000
TOTAL = BATCH * FIELDS

NUM_CORES = 2
NUM_SUBCORES = 16
NW = NUM_CORES * NUM_SUBCORES

CHUNK = 128
BCHUNKS = BATCH // CHUNK
BCHUNKS_PER_W = BCHUNKS // NW
N_TASKS = FIELDS * BCHUNKS_PER_W
SPAN = CHUNK * BCHUNKS_PER_W

_MESH = plsc.VectorSubcoreMesh(core_axis_name="c", subcore_axis_name="s")


@functools.partial(
    pl.kernel,
    mesh=_MESH,
    compiler_params=pltpu.CompilerParams(use_tc_tiling_on_sc=False,
                                         needs_layout_passes=False),
    out_type=jax.ShapeDtypeStruct((FIELDS, 8, BCHUNKS, 8 * CHUNK),
                                  jnp.float32),
    scratch_types=[
        pltpu.VMEM((FIELDS, SPAN), jnp.int32),
        [pltpu.VMEM((CHUNK, DIM), jnp.float32)] * 2,
        [pltpu.VMEM((8, 8 * CHUNK), jnp.float32)] * 2,
        [pltpu.SemaphoreType.DMA] * 2,
        [pltpu.SemaphoreType.DMA] * 2,
    ],
)
def _gather_kernel(idx_hbm, table, out, idx_v, gbufs, obufs, gsems, osems):
    wid = lax.axis_index("s") * NUM_CORES + lax.axis_index("c")
    iota = lax.iota(jnp.int32, 16)

    def stage(f, carry):
        pltpu.sync_copy(idx_hbm.at[pl.ds(f * BATCH + wid * SPAN, SPAN)],
                        idx_v.at[f])
        return carry

    lax.fori_loop(0, FIELDS, stage, 0, unroll=False)

    def task_fc(t):
        return t // BCHUNKS_PER_W, t % BCHUNKS_PER_W

    def gather(t, p):
        f, cc = task_fc(t)
        return pltpu.make_async_copy(
            table.at[idx_v.at[f, pl.ds(cc * CHUNK, CHUNK)]],
            gbufs[p], gsems[p])

    def store(t, p):
        f, cc = task_fc(t)
        bt = wid * BCHUNKS_PER_W + cc
        return pltpu.make_async_copy(
            obufs[p], out.at[f, :, bt, :], osems[p])

    row_vecs = [jg * 16 + iota for jg in range(8)]

    def select_transpose(p):
        gbuf, obuf = gbufs[p], obufs[p]

        @plsc.parallel_loop(0, 8, unroll=8)
        def _(db):
            d0 = db * 8
            for ds_ in range(8):
                col = jnp.full((16,), d0 + ds_, jnp.int32)
                for jg in range(8):
                    v = plsc.load_gather(gbuf, [row_vecs[jg], col])
                    obuf[db, pl.ds(ds_ * CHUNK + jg * 16, 16)] = v

    for p in range(2):
        gather(p, p).start()

    def task(g, carry):
        for p in range(2):
            t = 2 * g + p
            gather(t, p).wait()

            @pl.when(g >= 1)
            def _():
                store(t - 2, p).wait()

            select_transpose(p)
            store(t, p).start()

            @pl.when(g < N_TASKS // 2 - 1)
            def _():
                gather(t + 2, p).start()

        return carry

    lax.fori_loop(0, N_TASKS // 2, task, 0, unroll=False)
    for p in range(2):
        store(N_TASKS - 2 + p, p).wait()


def kernel(input_card, table):
    idx_flat = input_card.T.astype(jnp.int32).reshape(TOTAL)
    out4 = _gather_kernel(idx_flat, table)
    out5 = out4.reshape(FIELDS, 8, BCHUNKS, 8, CHUNK)
    return out5.transpose(2, 4, 0, 1, 3).reshape(BATCH, FIELDS, DIM)


# speedup vs baseline: 1.4645x; 1.4645x over previous
import functools

import jax
import jax.numpy as jnp
from jax import lax
from jax.experimental import pallas as pl
from jax.experimental.pallas import tpu as pltpu
from jax.experimental.pallas import tpu_sc as plsc

BATCH = 16384
FIELDS = 26
DIM = 64
VOCAB = 1000000
TOTAL = BATCH * FIELDS

NUM_CORES = 2
NUM_SUBCORES = 16
NW = NUM_CORES * NUM_SUBCORES

CHUNK = 128
BCHUNKS = BATCH // CHUNK
BCHUNKS_PER_W = BCHUNKS // NW
N_TASKS = FIELDS * BCHUNKS_PER_W
SPAN = CHUNK * BCHUNKS_PER_W

_MESH = plsc.VectorSubcoreMesh(core_axis_name="c", subcore_axis_name="s")


@functools.partial(
    pl.kernel,
    mesh=_MESH,
    compiler_params=pltpu.CompilerParams(use_tc_tiling_on_sc=False,
                                         needs_layout_passes=False),
    out_type=jax.ShapeDtypeStruct((FIELDS, 8, BCHUNKS, 8, CHUNK),
                                  jnp.float32),
    scratch_types=[
        pltpu.VMEM((FIELDS, SPAN), jnp.int32),
        [pltpu.VMEM((CHUNK, DIM), jnp.float32)] * 2,
        [pltpu.VMEM((8, 8, CHUNK + 1), jnp.float32)] * 2,
        [pltpu.SemaphoreType.DMA] * 2,
        [pltpu.SemaphoreType.DMA] * 2,
    ],
)
def _gather_kernel(idx_hbm, table, out, idx_v, gbufs, obufs, gsems, osems):
    wid = lax.axis_index("s") * NUM_CORES + lax.axis_index("c")
    iota = lax.iota(jnp.int32, 16)

    def stage(f, carry):
        pltpu.sync_copy(idx_hbm.at[pl.ds(f * BATCH + wid * SPAN, SPAN)],
                        idx_v.at[f])
        return carry

    lax.fori_loop(0, FIELDS, stage, 0, unroll=False)

    def task_fc(t):
        return t // BCHUNKS_PER_W, t % BCHUNKS_PER_W

    def gather(t, p):
        f, cc = task_fc(t)
        return pltpu.make_async_copy(
            table.at[idx_v.at[f, pl.ds(cc * CHUNK, CHUNK)]],
            gbufs[p], gsems[p])

    def store(t, p):
        f, cc = task_fc(t)
        bt = wid * BCHUNKS_PER_W + cc
        return pltpu.make_async_copy(
            obufs[p].at[:, :, pl.ds(0, CHUNK)], out.at[f, :, bt, :, :],
            osems[p])

    dt_vecs = [(jnp.int32(dg * 16) + iota) // 8 for dg in range(4)]
    ds_vecs = [(jnp.int32(dg * 16) + iota) % 8 for dg in range(4)]

    def select_transpose(p):
        gbuf, obuf = gbufs[p], obufs[p]

        @plsc.parallel_loop(0, CHUNK, unroll=4)
        def _(j):
            col = jnp.full((16,), j, jnp.int32)
            for dg in range(4):
                v = gbuf[j, pl.ds(dg * 16, 16)]
                plsc.store_scatter(obuf, [dt_vecs[dg], ds_vecs[dg], col], v)

    for p in range(2):
        gather(p, p).start()

    def task(g, carry):
        for p in range(2):
            t = 2 * g + p
            gather(t, p).wait()

            @pl.when(g >= 1)
            def _():
                store(t - 2, p).wait()

            select_transpose(p)
            store(t, p).start()

            @pl.when(g < N_TASKS // 2 - 1)
            def _():
                gather(t + 2, p).start()

        return carry

    lax.fori_loop(0, N_TASKS // 2, task, 0, unroll=False)
    for p in range(2):
        store(N_TASKS - 2 + p, p).wait()


def kernel(input_card, table):
    idx_flat = input_card.T.astype(jnp.int32).reshape(TOTAL)
    out5 = _gather_kernel(idx_flat, table)
    return out5.transpose(2, 4, 0, 1, 3).reshape(BATCH, FIELDS, DIM)


# speedup vs baseline: 1.5034x; 1.0266x over previous
import functools

import jax
import jax.numpy as jnp
from jax import lax
from jax.experimental import pallas as pl
from jax.experimental.pallas import tpu as pltpu
from jax.experimental.pallas import tpu_sc as plsc

BATCH = 16384
FIELDS = 26
DIM = 64
VOCAB = 1000000
TOTAL = BATCH * FIELDS

NUM_CORES = 2
NUM_SUBCORES = 16
NW = NUM_CORES * NUM_SUBCORES

CHUNK = 128
BCHUNKS = BATCH // CHUNK
BCHUNKS_PER_W = BCHUNKS // NW
N_TASKS = FIELDS * BCHUNKS_PER_W
SPAN = CHUNK * BCHUNKS_PER_W

_MESH = plsc.VectorSubcoreMesh(core_axis_name="c", subcore_axis_name="s")


@functools.partial(
    pl.kernel,
    mesh=_MESH,
    compiler_params=pltpu.CompilerParams(use_tc_tiling_on_sc=False,
                                         needs_layout_passes=False),
    out_type=jax.ShapeDtypeStruct((FIELDS, 8, BCHUNKS, 8, CHUNK),
                                  jnp.float32),
    scratch_types=[
        pltpu.VMEM((FIELDS, SPAN), jnp.int32),
        [pltpu.VMEM((CHUNK, DIM), jnp.float32)] * 4,
        [pltpu.VMEM((8, 8, CHUNK + 1), jnp.float32)] * 4,
        [pltpu.SemaphoreType.DMA] * 4,
        [pltpu.SemaphoreType.DMA] * 4,
    ],
)
def _gather_kernel(idx_hbm, table, out, idx_v, gbufs, obufs, gsems, osems):
    wid = lax.axis_index("s") * NUM_CORES + lax.axis_index("c")
    iota = lax.iota(jnp.int32, 16)

    def stage(f, carry):
        pltpu.sync_copy(idx_hbm.at[pl.ds(f * BATCH + wid * SPAN, SPAN)],
                        idx_v.at[f])
        return carry

    lax.fori_loop(0, FIELDS, stage, 0, unroll=False)

    def task_fc(t):
        return t // BCHUNKS_PER_W, t % BCHUNKS_PER_W

    def gather(t, p):
        f, cc = task_fc(t)
        return pltpu.make_async_copy(
            table.at[idx_v.at[f, pl.ds(cc * CHUNK, CHUNK)]],
            gbufs[p], gsems[p])

    def store(t, p):
        f, cc = task_fc(t)
        bt = wid * BCHUNKS_PER_W + cc
        return pltpu.make_async_copy(
            obufs[p].at[:, :, pl.ds(0, CHUNK)], out.at[f, :, bt, :, :],
            osems[p])

    dt_vecs = [(jnp.int32(dg * 16) + iota) // 8 for dg in range(4)]
    ds_vecs = [(jnp.int32(dg * 16) + iota) % 8 for dg in range(4)]

    def select_transpose(p):
        gbuf, obuf = gbufs[p], obufs[p]

        @plsc.parallel_loop(0, CHUNK, unroll=4)
        def _(j):
            col = jnp.full((16,), j, jnp.int32)
            for dg in range(4):
                v = gbuf[j, pl.ds(dg * 16, 16)]
                plsc.store_scatter(obuf, [dt_vecs[dg], ds_vecs[dg], col], v)

    NBUF = 4
    for p in range(NBUF):
        gather(p, p).start()

    def task(g, carry):
        for p in range(NBUF):
            t = NBUF * g + p
            gather(t, p).wait()

            @pl.when(g >= 1)
            def _():
                store(t - NBUF, p).wait()

            select_transpose(p)
            store(t, p).start()

            @pl.when(g < N_TASKS // NBUF - 1)
            def _():
                gather(t + NBUF, p).start()

        return carry

    lax.fori_loop(0, N_TASKS // NBUF, task, 0, unroll=False)
    for p in range(NBUF):
        store(N_TASKS - NBUF + p, p).wait()


def kernel(input_card, table):
    idx_flat = input_card.T.astype(jnp.int32).reshape(TOTAL)
    out5 = _gather_kernel(idx_flat, table)
    return out5.transpose(2, 4, 0, 1, 3).reshape(BATCH, FIELDS, DIM)
